# baseline (speedup 1.0000x reference)
import functools

import jax
import jax.numpy as jnp
from jax import lax
from jax.experimental import pallas as pl
from jax.experimental.pallas import tpu as pltpu
from jax.experimental.pallas import tpu_sc as plsc

N = 16384
C = 1000

_info = plsc.get_sparse_core_info()
_NC, _NS = _info.num_cores, _info.num_subcores
_NW = _NC * _NS
_CPW = N // _NW
_CR = 40
_NCHUNK = C // _CR
_NPAIR = _NCHUNK // 2
_NS16 = _CPW // 16
_NACC = 8


def _sc_loss_partials(yht, y):
    mesh = plsc.VectorSubcoreMesh(core_axis_name="c", subcore_axis_name="s")

    @functools.partial(
        pl.kernel,
        mesh=mesh,
        out_type=jax.ShapeDtypeStruct((_NW, 16), jnp.float32),
        scratch_types=[
            pltpu.VMEM((_CR, _CPW), jnp.float32),
            pltpu.VMEM((_CR, _CPW), jnp.float32),
            pltpu.VMEM((_CPW,), jnp.int32),
            pltpu.VMEM((16,), jnp.float32),
            pltpu.SemaphoreType.DMA,
            pltpu.SemaphoreType.DMA,
        ],
    )
    def k(yht_hbm, y_hbm, out_hbm, buf0, buf1, y_v, acc_v, sem0, sem1):
        wid = lax.axis_index("s") * _NC + lax.axis_index("c")
        col0 = wid * _CPW
        pltpu.sync_copy(y_hbm.at[pl.ds(col0, _CPW)], y_v)

        def start(ch, buf, sem):
            pltpu.async_copy(
                yht_hbm.at[pl.ds(ch * _CR, _CR), pl.ds(col0, _CPW)], buf, sem
            )

        def drain(buf, sem):
            pltpu.make_async_copy(
                yht_hbm.at[pl.ds(0, _CR), pl.ds(0, _CPW)], buf, sem
            ).wait()

        def consume(ch, buf, carry):
            r0 = ch * _CR

            def s_body(s, aa):
                aa = list(aa)
                cb = pl.multiple_of(s * 16, 16)
                rel = y_v[pl.ds(cb, 16)] - r0
                for rr in range(_CR):
                    x = buf[rr, pl.ds(cb, 16)]
                    aa[rr % _NACC] = aa[rr % _NACC] + jnp.where(
                        rel == rr, -x, x
                    )
                return tuple(aa)

            return lax.fori_loop(0, _NS16, s_body, carry)

        start(0, buf0, sem0)

        def pair_body(p, carry):
            ch0 = p * 2
            start(ch0 + 1, buf1, sem1)
            drain(buf0, sem0)
            carry = consume(ch0, buf0, carry)
            start(ch0 + 2, buf0, sem0)
            drain(buf1, sem1)
            carry = consume(ch0 + 1, buf1, carry)
            return carry

        zero = jnp.zeros((16,), jnp.float32)
        carry = lax.fori_loop(0, _NPAIR, pair_body, tuple([zero] * _NACC))
        drain(buf0, sem0)
        carry = consume(_NCHUNK - 1, buf0, carry)

        acc = carry[0]
        for a in carry[1:]:
            acc = acc + a
        acc_v[...] = acc
        pltpu.sync_copy(acc_v, out_hbm.at[wid])

    return k(yht, y)


def kernel(yh, y):
    partials = _sc_loss_partials(yh.T, y.astype(jnp.int32))
    return partials.sum()
